# Initial kernel scaffold; baseline (speedup 1.0000x reference)
import jax
import jax.numpy as jnp
from jax.experimental import pallas as pl


def kernel(predicted_locs, predicted_scores, boxes, labels, priors_cxcycz, threshold):
    raise NotImplementedError("write your pallas kernel here")


# speedup vs baseline: 25.7164x; 25.7164x over previous
import functools

import jax
import jax.numpy as jnp
from jax.experimental import pallas as pl
from jax.experimental.pallas import tpu as pltpu

_B, _P, _NOBJ, _NC = 8, 20000, 12, 2
_PPAD = 20480
_BIGIDX = 1e9


def _loss_kernel(locs_ref, scores_ref, boxes_ref, labels_ref, priors_ref,
                 thr_ref, conf_ref, loc_ref, acc_ref):
    b = pl.program_id(0)

    @pl.when(b == 0)
    def _init():
        acc_ref[0] = 0.0
        acc_ref[1] = 0.0
        acc_ref[2] = 0.0

    thr = thr_ref[0, 0]

    iota_p = jax.lax.broadcasted_iota(
        jnp.int32, (1, _PPAD), 1).astype(jnp.float32)
    valid = iota_p < float(_P)
    iota_obj = jax.lax.broadcasted_iota(
        jnp.int32, (_NOBJ, 1), 0).astype(jnp.float32)

    bx = boxes_ref[0]
    lb = labels_ref[0]
    pr = priors_ref[...]

    p_lo = [pr[d:d + 1, :] - pr[d + 3:d + 4, :] * 0.5 for d in range(3)]
    p_hi = [pr[d:d + 1, :] + pr[d + 3:d + 4, :] * 0.5 for d in range(3)]

    inter = jnp.float32(1.0)
    vol_a = jnp.float32(1.0)
    vol_b = jnp.float32(1.0)
    for d in range(3):
        a_lo = bx[:, d:d + 1]
        a_hi = bx[:, d + 3:d + 4]
        lo = jnp.maximum(a_lo, p_lo[d])
        hi = jnp.minimum(a_hi, p_hi[d])
        inter = inter * jnp.maximum(hi - lo, 0.0)
        vol_a = vol_a * (a_hi - a_lo)
        vol_b = vol_b * (p_hi[d] - p_lo[d])
    ov = inter / (vol_a + vol_b - inter)
    ov = jnp.where(valid, ov, -1.0)

    ov_best = jnp.max(ov, axis=0, keepdims=True)
    obj = jnp.min(jnp.where(ov == ov_best, iota_obj, _BIGIDX),
                  axis=0, keepdims=True)

    rowmax = jnp.max(ov, axis=1, keepdims=True)
    pfeo = jnp.min(jnp.where(ov == rowmax, iota_p, _BIGIDX),
                   axis=1, keepdims=True)
    hit = iota_p == pfeo
    j_over = jnp.max(jnp.where(hit, iota_obj, -1.0), axis=0,
                     keepdims=True)
    forced = j_over >= 0.0
    obj = jnp.where(forced, j_over, obj)
    ov_best = jnp.where(forced, 1.0, ov_best)

    sel = (iota_obj == obj).astype(jnp.float32)

    label = jnp.sum(sel * lb, axis=0, keepdims=True)
    label = jnp.where(ov_best < thr, 0.0, label)
    label = jnp.where(valid, label, 0.0)
    pos = (label > 0.0).astype(jnp.float32)

    diffsum = jnp.zeros((1, _PPAD), jnp.float32)
    for d in range(3):
        cc = jnp.sum(sel * ((bx[:, d:d + 1] + bx[:, d + 3:d + 4]) * 0.5),
                     axis=0, keepdims=True)
        cs = jnp.sum(sel * (bx[:, d + 3:d + 4] - bx[:, d:d + 1]),
                     axis=0, keepdims=True)
        p_ctr = pr[d:d + 1, :]
        p_sz = pr[d + 3:d + 4, :]
        g_ctr = (cc - p_ctr) / (p_sz * 0.1)
        g_sz = jnp.log(cs / p_sz) * 5.0
        diffsum += jnp.abs(locs_ref[0, d:d + 1, :] - g_ctr)
        diffsum += jnp.abs(locs_ref[0, d + 3:d + 4, :] - g_sz)

    acc_ref[0] += jnp.sum(diffsum * pos)
    acc_ref[1] += jnp.sum(pos)

    i = scores_ref[0, 1:2, :]
    t = (label == 1.0).astype(jnp.float32)
    bce = jnp.maximum(i, 0.0) - i * t + jnp.log1p(jnp.exp(-jnp.abs(i)))
    x = -i * (t * 2.0 - 1.0)
    log_sig = jnp.minimum(x, 0.0) - jnp.log1p(jnp.exp(-jnp.abs(x)))
    focal = (t * 0.25 + (1.0 - t) * 0.75) * jnp.exp(log_sig * 2.0) * bce
    acc_ref[2] += jnp.sum(jnp.where(valid, focal, 0.0))

    @pl.when(b == _B - 1)
    def _finalize():
        n_pos = acc_ref[1]
        denom = jnp.maximum(n_pos * 6.0, 1.0)
        loc_val = jnp.where(n_pos == 0.0, 0.0, acc_ref[0] / denom)
        loc_ref[...] = jnp.full((1, 1), loc_val, jnp.float32)
        conf_ref[...] = jnp.full((1, 1), acc_ref[2] / jnp.float32(_B * _P),
                                 jnp.float32)


@jax.jit
def _run(predicted_locs, predicted_scores, boxes, labels, priors_cxcycz,
         threshold):
    pad = _PPAD - _P
    locs = jnp.pad(predicted_locs.transpose(0, 2, 1), ((0, 0), (0, 0), (0, pad)))
    scores = jnp.pad(predicted_scores.transpose(0, 2, 1),
                     ((0, 0), (0, 0), (0, pad)))
    priors = jnp.pad(priors_cxcycz.T, ((0, 0), (0, pad)))
    priors = priors.at[3:, _P:].set(1.0)
    lab = labels.astype(jnp.float32)[..., None]
    thr = jnp.asarray(threshold, jnp.float32).reshape(1, 1)

    conf, loc = pl.pallas_call(
        _loss_kernel,
        grid=(_B,),
        in_specs=[
            pl.BlockSpec((1, 6, _PPAD), lambda b: (b, 0, 0)),
            pl.BlockSpec((1, _NC, _PPAD), lambda b: (b, 0, 0)),
            pl.BlockSpec((1, _NOBJ, 6), lambda b: (b, 0, 0)),
            pl.BlockSpec((1, _NOBJ, 1), lambda b: (b, 0, 0)),
            pl.BlockSpec((6, _PPAD), lambda b: (0, 0)),
            pl.BlockSpec(memory_space=pltpu.SMEM),
        ],
        out_specs=[
            pl.BlockSpec((1, 1), lambda b: (0, 0)),
            pl.BlockSpec((1, 1), lambda b: (0, 0)),
        ],
        out_shape=[
            jax.ShapeDtypeStruct((1, 1), jnp.float32),
            jax.ShapeDtypeStruct((1, 1), jnp.float32),
        ],
        scratch_shapes=[pltpu.SMEM((3,), jnp.float32)],
    )(locs, scores, boxes, lab, priors, thr)
    return conf[0, 0], loc[0, 0]


def kernel(predicted_locs, predicted_scores, boxes, labels, priors_cxcycz,
           threshold):
    return _run(predicted_locs, predicted_scores, boxes, labels,
                priors_cxcycz, threshold)
